# baseline (device time: 18261 ns/iter reference)
import jax
import jax.numpy as jnp
from jax import lax
from jax.experimental import pallas as pl
from jax.experimental.pallas import tpu as pltpu

N_DEV = 4
B = 2
SQ = 256
D_MODEL = 512
HQ = 4
DH = 64
HD = HQ * DH
BH = B * HQ
SKV_SHARD = 256
BLK = 64
SCALE = 0.125


def kernel(x, Wq, K_ext, V_ext, Wo):
    def body(x_hbm, wq_hbm, k_hbm, v_hbm, wo_hbm, out_hbm,
             xv, wqv, kv, vv, wov, outv,
             ctxbuf, statbuf, copy_sems, out_sems,
             csend, crecv, ssend, srecv):
        my = lax.axis_index("i")

        cp_x = pltpu.make_async_copy(x_hbm, xv, copy_sems.at[0])
        cp_wq = pltpu.make_async_copy(wq_hbm, wqv, copy_sems.at[1])
        cp_k = pltpu.make_async_copy(k_hbm, kv, copy_sems.at[2])
        cp_v = pltpu.make_async_copy(v_hbm, vv, copy_sems.at[3])
        cp_wo = pltpu.make_async_copy(wo_hbm, wov, copy_sems.at[4])
        for cp in (cp_x, cp_wq, cp_k, cp_v, cp_wo):
            cp.start()

        barrier_sem = pltpu.get_barrier_semaphore()
        for j in range(1, N_DEV):
            pl.semaphore_signal(
                barrier_sem, inc=1,
                device_id=((my + j) % N_DEV,),
                device_id_type=pl.DeviceIdType.MESH,
            )
        pl.semaphore_wait(barrier_sem, N_DEV - 1)

        qb = lax.broadcasted_iota(jnp.int32, (SQ, SKV_SHARD), 0) // BLK
        kbg = (my * SKV_SHARD
               + lax.broadcasted_iota(jnp.int32, (SQ, SKV_SHARD), 1)) // BLK
        mask = (qb == kbg) | (kbg == 0) | ((qb + kbg) % 3 == 0)

        cp_wq.wait()
        wq = wqv[...].astype(jnp.bfloat16)
        cp_x.wait()
        cp_k.wait()
        cp_v.wait()

        ctx_rdmas = [[None] * (N_DEV - 1) for _ in range(B)]
        stat_cols = []
        for b in range(B):
            xb = xv[b].astype(jnp.bfloat16)
            q = jnp.dot(xb, wq, preferred_element_type=jnp.float32)
            kloc = kv[b].reshape(SKV_SHARD, HD).astype(jnp.bfloat16)
            vloc = vv[b].reshape(SKV_SHARD, HD).astype(jnp.bfloat16)
            ctx_heads = []
            for h in range(HQ):
                qh = q[:, h * DH:(h + 1) * DH].astype(jnp.bfloat16)
                kh = kloc[:, h * DH:(h + 1) * DH]
                vh = vloc[:, h * DH:(h + 1) * DH]
                s = lax.dot_general(
                    qh, kh, (((1,), (1,)), ((), ())),
                    preferred_element_type=jnp.float32,
                ) * SCALE
                s = jnp.where(mask, s, -1e9)
                m = jnp.max(s, axis=1, keepdims=True)
                w = jnp.exp(s - m)
                l = jnp.sum(w, axis=1, keepdims=True)
                ctx_heads.append(
                    jnp.dot(w.astype(jnp.bfloat16), vh,
                            preferred_element_type=jnp.float32) / l
                )
                stat_cols.append((m, l))
            ctxbuf[my * B + b] = jnp.concatenate(ctx_heads, axis=1).astype(
                jnp.bfloat16)
            for j in range(N_DEV - 1):
                tgt = (my + 1 + j) % N_DEV
                rc = pltpu.make_async_remote_copy(
                    src_ref=ctxbuf.at[my * B + b],
                    dst_ref=ctxbuf.at[my * B + b],
                    send_sem=csend.at[j * B + b],
                    recv_sem=crecv.at[j * B + b],
                    device_id=(tgt,), device_id_type=pl.DeviceIdType.MESH,
                )
                rc.start()
                ctx_rdmas[b][j] = rc

        m_cols = jnp.concatenate([c[0] for c in stat_cols], axis=1)
        l_cols = jnp.concatenate([c[1] for c in stat_cols], axis=1)
        statbuf[my] = jnp.stack([m_cols.T, l_cols.T])
        stat_rdmas = []
        for j in range(N_DEV - 1):
            tgt = (my + 1 + j) % N_DEV
            rs = pltpu.make_async_remote_copy(
                src_ref=statbuf.at[my], dst_ref=statbuf.at[my],
                send_sem=ssend.at[j], recv_sem=srecv.at[j],
                device_id=(tgt,), device_id_type=pl.DeviceIdType.MESH,
            )
            rs.start()
            stat_rdmas.append(rs)

        for rs in stat_rdmas:
            rs.wait()
        stats = statbuf[...]
        statsT = jnp.transpose(stats, (0, 1, 3, 2))
        m_all = statsT[:, 0]
        l_all = statsT[:, 1]
        M = jnp.max(m_all, axis=0)
        wj = l_all * jnp.exp(m_all - M[None])
        coef = wj / jnp.sum(wj, axis=0)[None]

        S = (lax.broadcasted_iota(jnp.int32, (HQ, HD), 1) // DH
             == lax.broadcasted_iota(jnp.int32, (HQ, HD), 0)
             ).astype(jnp.float32)

        cp_wo.wait()
        wo = wov[...].astype(jnp.bfloat16)
        out_cps = []
        for b in range(B):
            for rc in ctx_rdmas[b]:
                rc.wait()
            acc = jnp.zeros((SQ, HD), jnp.float32)
            for slot in range(N_DEV):
                coefw = jnp.dot(coef[slot][:, b * HQ:(b + 1) * HQ], S,
                                preferred_element_type=jnp.float32)
                acc = acc + coefw * ctxbuf[slot * B + b].astype(jnp.float32)
            outv[b] = jnp.dot(acc.astype(jnp.bfloat16), wo,
                              preferred_element_type=jnp.float32
                              ).astype(jnp.bfloat16)
            cp_o = pltpu.make_async_copy(outv.at[b], out_hbm.at[b],
                                         out_sems.at[b])
            cp_o.start()
            out_cps.append(cp_o)
        for cp_o in out_cps:
            cp_o.wait()

    return pl.pallas_call(
        body,
        out_shape=jax.ShapeDtypeStruct((B, SQ, D_MODEL), jnp.bfloat16),
        in_specs=[pl.BlockSpec(memory_space=pl.ANY)] * 5,
        out_specs=pl.BlockSpec(memory_space=pl.ANY),
        scratch_shapes=[
            pltpu.VMEM((B, SQ, D_MODEL), jnp.float32),
            pltpu.VMEM((D_MODEL, HD), jnp.float32),
            pltpu.VMEM((B, SKV_SHARD, HQ, DH), jnp.float32),
            pltpu.VMEM((B, SKV_SHARD, HQ, DH), jnp.float32),
            pltpu.VMEM((HD, D_MODEL), jnp.float32),
            pltpu.VMEM((B, SQ, D_MODEL), jnp.bfloat16),
            pltpu.VMEM((N_DEV * B, SQ, HD), jnp.bfloat16),
            pltpu.VMEM((N_DEV, 2, BH, SQ), jnp.float32),
            pltpu.SemaphoreType.DMA((5,)),
            pltpu.SemaphoreType.DMA((B,)),
            pltpu.SemaphoreType.DMA(((N_DEV - 1) * B,)),
            pltpu.SemaphoreType.DMA(((N_DEV - 1) * B,)),
            pltpu.SemaphoreType.DMA((N_DEV - 1,)),
            pltpu.SemaphoreType.DMA((N_DEV - 1,)),
        ],
        compiler_params=pltpu.CompilerParams(collective_id=0),
    )(x, Wq, K_ext, V_ext, Wo)
